# baseline (speedup 1.0000x reference)
import functools

import jax
import jax.numpy as jnp
from jax import lax
from jax.experimental import pallas as pl
from jax.experimental.pallas import tpu as pltpu
from jax.experimental.pallas import tpu_sc as plsc

_D = 64
_N = 50000
_HALF = 25088
_NPAD = 2 * _HALF
_GARB = 16
_ACC = _HALF + _GARB
_EPAD = 802816
_IDXROWS = _EPAD // 128
_TILES = 16
_RPT = _IDXROWS // _TILES
_K = 8
_BATCH_E = _K * 128
_NB = _RPT // _K
_CPT = _HALF // _TILES
_REG = 50 * _BATCH_E
_PE = _TILES * _REG
_BUF = 2064


def _part_body(src, dst, w, psrc, pdst, pw, cnts, src_in, dst_in, w_in,
               bsrc, bdst, bw, cnt_v):
    c = lax.axis_index("c")
    s = lax.axis_index("s")
    half_base = c * _HALF
    lane = lax.iota(jnp.int32, 16)
    row0 = s * _RPT
    reg0 = s * _REG

    def _flush(cnt, off):
        pltpu.sync_copy(bsrc.at[pl.ds(0, _BATCH_E)],
                        psrc.at[c, pl.ds(reg0 + off * _BATCH_E, _BATCH_E)])
        pltpu.sync_copy(bdst.at[pl.ds(0, _BATCH_E)],
                        pdst.at[c, pl.ds(reg0 + off * _BATCH_E, _BATCH_E)])
        pltpu.sync_copy(bw.at[pl.ds(0, _BATCH_E)],
                        pw.at[c, pl.ds(reg0 + off * _BATCH_E, _BATCH_E)])

    def _batch(bi, carry):
        base = row0 + bi * _K
        pltpu.sync_copy(src.at[pl.ds(base, _K)], src_in)
        pltpu.sync_copy(dst.at[pl.ds(base, _K)], dst_in)
        pltpu.sync_copy(w.at[pl.ds(base * 128, _BATCH_E)], w_in)
        cnt, off = carry
        for j in range(_K):
            for g in range(8):
                sv = src_in[j, pl.ds(g * 16, 16)]
                dv = dst_in[j, pl.ds(g * 16, 16)]
                wv = w_in[pl.ds(j * 128 + g * 16, 16)]
                loc = dv - half_base
                keep = (loc >= 0) & (loc < _HALF)
                key = jnp.where(keep, lane, lane + 16)
                _, sv_s = plsc.sort_key_val(key, sv)
                _, loc_s = plsc.sort_key_val(key, loc)
                _, wv_s = plsc.sort_key_val(key, wv)
                bsrc[pl.ds(cnt, 16)] = sv_s
                bdst[pl.ds(cnt, 16)] = loc_s
                bw[pl.ds(cnt, 16)] = wv_s
                npc = plsc.all_reduce_population_count(keep)
                cnt = cnt + npc[0]

            full = cnt >= _BATCH_E

            @pl.when(full)
            def _do_flush():
                _flush(cnt, off)
                for gg in range(9):
                    tv = bsrc[pl.ds(_BATCH_E + gg * 16, 16)]
                    bsrc[pl.ds(gg * 16, 16)] = tv
                    tv = bdst[pl.ds(_BATCH_E + gg * 16, 16)]
                    bdst[pl.ds(gg * 16, 16)] = tv
                    tv = bw[pl.ds(_BATCH_E + gg * 16, 16)]
                    bw[pl.ds(gg * 16, 16)] = tv

            cnt = jnp.where(full, cnt - _BATCH_E, cnt)
            off = jnp.where(full, off + 1, off)
        return (cnt, off)

    cnt, off = lax.fori_loop(0, _NB, _batch,
                             (jnp.int32(0), jnp.int32(0)))

    def _gfill(g, _):
        p = cnt + g * 16
        bsrc[pl.ds(p, 16)] = jnp.zeros((16,), jnp.int32)
        bdst[pl.ds(p, 16)] = _HALF + lane
        bw[pl.ds(p, 16)] = jnp.zeros((16,), jnp.float32)
        return 0

    lax.fori_loop(0, _BATCH_E // 16, _gfill, 0)
    _flush(cnt, off)
    nb = off + 1
    cnt_v[...] = jnp.full((16,), 1, jnp.int32) * nb
    pltpu.sync_copy(cnt_v, cnts.at[c, s])


_partition = functools.partial(
    pl.kernel,
    mesh=plsc.VectorSubcoreMesh(core_axis_name="c", subcore_axis_name="s"),
    out_type=(
        jax.ShapeDtypeStruct((2, _PE), jnp.int32),
        jax.ShapeDtypeStruct((2, _PE), jnp.int32),
        jax.ShapeDtypeStruct((2, _PE), jnp.float32),
        jax.ShapeDtypeStruct((2, _TILES, 16), jnp.int32),
    ),
    compiler_params=pltpu.CompilerParams(use_tc_tiling_on_sc=False,
                                         needs_layout_passes=False),
    scratch_types=[
        pltpu.VMEM((_K, 128), jnp.int32),
        pltpu.VMEM((_K, 128), jnp.int32),
        pltpu.VMEM((_BATCH_E,), jnp.float32),
        pltpu.VMEM((_BUF,), jnp.int32),
        pltpu.VMEM((_BUF,), jnp.int32),
        pltpu.VMEM((_BUF,), jnp.float32),
        pltpu.VMEM((16,), jnp.int32),
    ],
)(_part_body)


def _layer_body(emb, psrc, pdst, pw, cnts, out, src_v, dloc_v, w_v, cnt_v,
                rows_a, rows_b, acc, sem):
    c = lax.axis_index("c")
    s = lax.axis_index("s")
    half_base = c * _HALF
    zv = jnp.zeros((16,), jnp.float32)

    def _zrow(i, _):
        for b in range(4):
            rows_a[i, pl.ds(b * 16, 16)] = zv
        return 0

    lax.fori_loop(0, 128, _zrow, 0)
    lb = s * _CPT
    for t in range(_CPT // 128):
        pltpu.sync_copy(rows_a, acc.at[pl.ds(lb + t * 128, 128)])
    if _CPT % 128:
        pltpu.sync_copy(rows_a.at[pl.ds(0, _CPT % 128)],
                        acc.at[pl.ds(lb + _CPT - _CPT % 128, _CPT % 128)])

    @pl.when(s == 0)
    def _zero_garbage():
        pltpu.sync_copy(rows_a.at[pl.ds(0, _GARB)], acc.at[pl.ds(_HALF, _GARB)])

    plsc.subcore_barrier()

    pltpu.sync_copy(cnts.at[c, s], cnt_v)
    nb = jnp.minimum(cnt_v[pl.ds(0, 16)][0], _REG // _BATCH_E)
    reg0 = s * _REG
    bufs = (rows_a, rows_b)

    def _scale(buf, j):
        def _grp(g, _):
            wv16 = w_v[pl.ds(j * 128 + g * 16, 16)]
            e0 = g * 16
            for k in range(16):
                wk = jnp.full((16,), wv16[k], jnp.float32)
                for b in range(4):
                    buf[e0 + k, pl.ds(b * 16, 16)] = (
                        buf[e0 + k, pl.ds(b * 16, 16)] * wk)
            return 0

        lax.fori_loop(0, 8, _grp, 0)

    def _batch(ci, _):
        base = reg0 + ci * _BATCH_E
        pltpu.sync_copy(pw.at[c, pl.ds(base, _BATCH_E)], w_v)
        for j in range(_K):
            pltpu.sync_copy(psrc.at[c, pl.ds(base + j * 128, 128)],
                            src_v.at[j])
            pltpu.sync_copy(pdst.at[c, pl.ds(base + j * 128, 128)],
                            dloc_v.at[j])
        cp = pltpu.async_copy(emb.at[src_v.at[0]], bufs[0], sem)
        for j in range(_K):
            buf = bufs[j % 2]
            if j + 1 < _K:
                cp_next = pltpu.async_copy(emb.at[src_v.at[j + 1]],
                                           bufs[(j + 1) % 2], sem)
            cp.wait()
            _scale(buf, j)
            pltpu.sync_copy(buf, acc.at[dloc_v.at[j]], add=True)
            if j + 1 < _K:
                cp = cp_next
        return 0

    lax.fori_loop(0, nb, _batch, 0)

    plsc.subcore_barrier()
    pltpu.sync_copy(acc.at[pl.ds(lb, _CPT)],
                    out.at[pl.ds(half_base + lb, _CPT)])


_layer = functools.partial(
    pl.kernel,
    mesh=plsc.VectorSubcoreMesh(core_axis_name="c", subcore_axis_name="s"),
    out_type=jax.ShapeDtypeStruct((_NPAD, _D), jnp.float32),
    compiler_params=pltpu.CompilerParams(use_tc_tiling_on_sc=False),
    scratch_types=[
        pltpu.VMEM((_K, 128), jnp.int32),
        pltpu.VMEM((_K, 128), jnp.int32),
        pltpu.VMEM((_BATCH_E,), jnp.float32),
        pltpu.VMEM((16,), jnp.int32),
        pltpu.VMEM((128, _D), jnp.float32),
        pltpu.VMEM((128, _D), jnp.float32),
        pltpu.VMEM_SHARED((_ACC, _D), jnp.float32),
        pltpu.SemaphoreType.DMA,
    ],
)(_layer_body)


def _mean_body(a, b, c, d, o):
    o[...] = (a[...] + b[...] + c[...] + d[...]) * 0.25


def _mean4(e0, e1, e2, e3):
    bs = pl.BlockSpec((1024, _D), lambda i: (i, 0))
    return pl.pallas_call(
        _mean_body,
        grid=(_NPAD // 1024,),
        in_specs=[bs] * 4,
        out_specs=bs,
        out_shape=jax.ShapeDtypeStruct((_NPAD, _D), jnp.float32),
    )(e0, e1, e2, e3)


def kernel(all_users, all_items, edge_index, edge_weight):
    n_users = all_users.shape[0]
    emb0 = jnp.concatenate([all_users, all_items], axis=0)
    emb0 = jnp.pad(emb0, ((0, _NPAD - _N), (0, 0)))
    src = edge_index[0].astype(jnp.int32)
    dst = edge_index[1].astype(jnp.int32)
    w = edge_weight.astype(jnp.float32)
    pad = _EPAD - src.shape[0]
    src = jnp.pad(src, (0, pad)).reshape(_IDXROWS, 128)
    dst = jnp.pad(dst, (0, pad), constant_values=_NPAD).reshape(_IDXROWS, 128)
    w = jnp.pad(w, (0, pad))
    psrc, pdst, pw, cnts = _partition(src, dst, w)
    e1 = _layer(emb0, psrc, pdst, pw, cnts)
    e2 = _layer(e1, psrc, pdst, pw, cnts)
    e3 = _layer(e2, psrc, pdst, pw, cnts)
    out = _mean4(emb0, e1, e2, e3)
    return (out[:n_users], out[n_users:_N])


# speedup vs baseline: 1.1723x; 1.1723x over previous
import functools

import jax
import jax.numpy as jnp
from jax import lax
from jax.experimental import pallas as pl
from jax.experimental.pallas import tpu as pltpu
from jax.experimental.pallas import tpu_sc as plsc

_D = 64
_N = 50000
_HALF = 25088
_NPAD = 2 * _HALF
_GARB = 16
_ACC = _HALF + _GARB
_EPAD = 802816
_IDXROWS = _EPAD // 128
_TILES = 16
_RPT = _IDXROWS // _TILES
_K = 8
_BATCH_E = _K * 128
_NB = _RPT // _K
_CPT = _HALF // _TILES
_REG = 50 * _BATCH_E
_PE = _TILES * _REG
_BUF = 2064


def _part_body(src, dst, w, psrc, pdst, pw, cnts, src_in, dst_in, w_in,
               bsrc, bdst, bw, cnt_v):
    c = lax.axis_index("c")
    s = lax.axis_index("s")
    half_base = c * _HALF
    lane = lax.iota(jnp.int32, 16)
    row0 = s * _RPT
    reg0 = s * _REG

    def _flush(cnt, off):
        pltpu.sync_copy(bsrc.at[pl.ds(0, _BATCH_E)],
                        psrc.at[c, pl.ds(reg0 + off * _BATCH_E, _BATCH_E)])
        pltpu.sync_copy(bdst.at[pl.ds(0, _BATCH_E)],
                        pdst.at[c, pl.ds(reg0 + off * _BATCH_E, _BATCH_E)])
        pltpu.sync_copy(bw.at[pl.ds(0, _BATCH_E)],
                        pw.at[c, pl.ds(reg0 + off * _BATCH_E, _BATCH_E)])

    def _batch(bi, carry):
        base = row0 + bi * _K
        pltpu.sync_copy(src.at[pl.ds(base, _K)], src_in)
        pltpu.sync_copy(dst.at[pl.ds(base, _K)], dst_in)
        pltpu.sync_copy(w.at[pl.ds(base * 128, _BATCH_E)], w_in)
        cnt, off = carry
        for j in range(_K):
            for g in range(8):
                sv = src_in[j, pl.ds(g * 16, 16)]
                dv = dst_in[j, pl.ds(g * 16, 16)]
                wv = w_in[pl.ds(j * 128 + g * 16, 16)]
                loc = dv - half_base
                keep = (loc >= 0) & (loc < _HALF)
                key = jnp.where(keep, lane, lane + 16)
                _, sv_s = plsc.sort_key_val(key, sv)
                _, loc_s = plsc.sort_key_val(key, loc)
                _, wv_s = plsc.sort_key_val(key, wv)
                bsrc[pl.ds(cnt, 16)] = sv_s
                bdst[pl.ds(cnt, 16)] = loc_s
                bw[pl.ds(cnt, 16)] = wv_s
                npc = plsc.all_reduce_population_count(keep)
                cnt = cnt + npc[0]

            full = cnt >= _BATCH_E

            @pl.when(full)
            def _do_flush():
                _flush(cnt, off)
                for gg in range(9):
                    tv = bsrc[pl.ds(_BATCH_E + gg * 16, 16)]
                    bsrc[pl.ds(gg * 16, 16)] = tv
                    tv = bdst[pl.ds(_BATCH_E + gg * 16, 16)]
                    bdst[pl.ds(gg * 16, 16)] = tv
                    tv = bw[pl.ds(_BATCH_E + gg * 16, 16)]
                    bw[pl.ds(gg * 16, 16)] = tv

            cnt = jnp.where(full, cnt - _BATCH_E, cnt)
            off = jnp.where(full, off + 1, off)
        return (cnt, off)

    cnt, off = lax.fori_loop(0, _NB, _batch,
                             (jnp.int32(0), jnp.int32(0)))

    def _gfill(g, _):
        p = cnt + g * 16
        bsrc[pl.ds(p, 16)] = jnp.zeros((16,), jnp.int32)
        bdst[pl.ds(p, 16)] = _HALF + lane
        bw[pl.ds(p, 16)] = jnp.zeros((16,), jnp.float32)
        return 0

    lax.fori_loop(0, _BATCH_E // 16, _gfill, 0)
    _flush(cnt, off)
    nb = off + 1
    cnt_v[...] = jnp.full((16,), 1, jnp.int32) * nb
    pltpu.sync_copy(cnt_v, cnts.at[c, s])


_partition = functools.partial(
    pl.kernel,
    mesh=plsc.VectorSubcoreMesh(core_axis_name="c", subcore_axis_name="s"),
    out_type=(
        jax.ShapeDtypeStruct((2, _PE), jnp.int32),
        jax.ShapeDtypeStruct((2, _PE), jnp.int32),
        jax.ShapeDtypeStruct((2, _PE), jnp.float32),
        jax.ShapeDtypeStruct((2, _TILES, 16), jnp.int32),
    ),
    compiler_params=pltpu.CompilerParams(use_tc_tiling_on_sc=False,
                                         needs_layout_passes=False),
    scratch_types=[
        pltpu.VMEM((_K, 128), jnp.int32),
        pltpu.VMEM((_K, 128), jnp.int32),
        pltpu.VMEM((_BATCH_E,), jnp.float32),
        pltpu.VMEM((_BUF,), jnp.int32),
        pltpu.VMEM((_BUF,), jnp.int32),
        pltpu.VMEM((_BUF,), jnp.float32),
        pltpu.VMEM((16,), jnp.int32),
    ],
)(_part_body)


def _layer_body(emb, psrc, pdst, pw, cnts, out, src_v, dloc_v, w_v, cnt_v,
                rows_a, rows_b, acc, sem, sem2):
    c = lax.axis_index("c")
    s = lax.axis_index("s")
    half_base = c * _HALF
    zv = jnp.zeros((16,), jnp.float32)

    def _zrow(i, _):
        for b in range(4):
            rows_a[i, pl.ds(b * 16, 16)] = zv
        return 0

    lax.fori_loop(0, 128, _zrow, 0)
    lb = s * _CPT
    for t in range(_CPT // 128):
        pltpu.sync_copy(rows_a, acc.at[pl.ds(lb + t * 128, 128)])
    if _CPT % 128:
        pltpu.sync_copy(rows_a.at[pl.ds(0, _CPT % 128)],
                        acc.at[pl.ds(lb + _CPT - _CPT % 128, _CPT % 128)])

    @pl.when(s == 0)
    def _zero_garbage():
        pltpu.sync_copy(rows_a.at[pl.ds(0, _GARB)], acc.at[pl.ds(_HALF, _GARB)])

    plsc.subcore_barrier()

    pltpu.sync_copy(cnts.at[c, s], cnt_v)
    nb = jnp.minimum(cnt_v[pl.ds(0, 16)][0], _REG // _BATCH_E)
    reg0 = s * _REG
    bufs = (rows_a, rows_b)

    def _scale(buf, j):
        def _grp(g, _):
            wv16 = w_v[pl.ds(j * 128 + g * 16, 16)]
            e0 = g * 16
            for k in range(16):
                wk = jnp.full((16,), wv16[k], jnp.float32)
                for b in range(4):
                    buf[e0 + k, pl.ds(b * 16, 16)] = (
                        buf[e0 + k, pl.ds(b * 16, 16)] * wk)
            return 0

        lax.fori_loop(0, 8, _grp, 0)

    def _batch(ci, _):
        base = reg0 + ci * _BATCH_E
        idx_cps = [pltpu.async_copy(pw.at[c, pl.ds(base, _BATCH_E)], w_v,
                                    sem2)]
        for j in range(_K):
            idx_cps.append(pltpu.async_copy(
                psrc.at[c, pl.ds(base + j * 128, 128)], src_v.at[j], sem2))
            idx_cps.append(pltpu.async_copy(
                pdst.at[c, pl.ds(base + j * 128, 128)], dloc_v.at[j], sem2))
        for icp in idx_cps:
            icp.wait()
        cp = pltpu.async_copy(emb.at[src_v.at[0]], bufs[0], sem)
        for j in range(_K):
            buf = bufs[j % 2]
            if j + 1 < _K:
                cp_next = pltpu.async_copy(emb.at[src_v.at[j + 1]],
                                           bufs[(j + 1) % 2], sem)
            cp.wait()
            _scale(buf, j)
            pltpu.sync_copy(buf, acc.at[dloc_v.at[j]], add=True)
            if j + 1 < _K:
                cp = cp_next
        return 0

    lax.fori_loop(0, nb, _batch, 0)

    plsc.subcore_barrier()
    pltpu.sync_copy(acc.at[pl.ds(lb, _CPT)],
                    out.at[pl.ds(half_base + lb, _CPT)])


_layer = functools.partial(
    pl.kernel,
    mesh=plsc.VectorSubcoreMesh(core_axis_name="c", subcore_axis_name="s"),
    out_type=jax.ShapeDtypeStruct((_NPAD, _D), jnp.float32),
    compiler_params=pltpu.CompilerParams(use_tc_tiling_on_sc=False),
    scratch_types=[
        pltpu.VMEM((_K, 128), jnp.int32),
        pltpu.VMEM((_K, 128), jnp.int32),
        pltpu.VMEM((_BATCH_E,), jnp.float32),
        pltpu.VMEM((16,), jnp.int32),
        pltpu.VMEM((128, _D), jnp.float32),
        pltpu.VMEM((128, _D), jnp.float32),
        pltpu.VMEM_SHARED((_ACC, _D), jnp.float32),
        pltpu.SemaphoreType.DMA,
        pltpu.SemaphoreType.DMA,
    ],
)(_layer_body)


def _mean_body(a, b, c, d, o):
    o[...] = (a[...] + b[...] + c[...] + d[...]) * 0.25


def _mean4(e0, e1, e2, e3):
    bs = pl.BlockSpec((1024, _D), lambda i: (i, 0))
    return pl.pallas_call(
        _mean_body,
        grid=(_NPAD // 1024,),
        in_specs=[bs] * 4,
        out_specs=bs,
        out_shape=jax.ShapeDtypeStruct((_NPAD, _D), jnp.float32),
    )(e0, e1, e2, e3)


def kernel(all_users, all_items, edge_index, edge_weight):
    n_users = all_users.shape[0]
    emb0 = jnp.concatenate([all_users, all_items], axis=0)
    emb0 = jnp.pad(emb0, ((0, _NPAD - _N), (0, 0)))
    src = edge_index[0].astype(jnp.int32)
    dst = edge_index[1].astype(jnp.int32)
    w = edge_weight.astype(jnp.float32)
    pad = _EPAD - src.shape[0]
    src = jnp.pad(src, (0, pad)).reshape(_IDXROWS, 128)
    dst = jnp.pad(dst, (0, pad), constant_values=_NPAD).reshape(_IDXROWS, 128)
    w = jnp.pad(w, (0, pad))
    psrc, pdst, pw, cnts = _partition(src, dst, w)
    e1 = _layer(emb0, psrc, pdst, pw, cnts)
    e2 = _layer(e1, psrc, pdst, pw, cnts)
    e3 = _layer(e2, psrc, pdst, pw, cnts)
    out = _mean4(emb0, e1, e2, e3)
    return (out[:n_users], out[n_users:_N])
